# Initial kernel scaffold; baseline (speedup 1.0000x reference)
import functools

import jax
import jax.numpy as jnp
from jax import lax
from jax.experimental import pallas as pl
from jax.experimental.pallas import tpu as pltpu
from jax.experimental.pallas import tpu_sc as plsc

N = 10000
E = 160000
H = 256
L = 128
ED = 16
NC = 3
G = 64

F = H // 2
SC_TILES = 16
EDGES_PER_TILE = E // SC_TILES
EB = 80
STEPS = EDGES_PER_TILE // EB
ROWS_PER_TILE = N // SC_TILES


def _sc_conv(x_lo, x_hi, ea_lo, ea_hi, src, dst):
    mesh = plsc.VectorSubcoreMesh(core_axis_name="c", subcore_axis_name="s",
                                  num_cores=2, num_subcores=16)

    @functools.partial(
        pl.kernel,
        out_type=[jax.ShapeDtypeStruct((N, F), jnp.float32),
                  jax.ShapeDtypeStruct((N, F), jnp.float32)],
        mesh=mesh,
        scratch_types=[
            pltpu.VMEM((EB,), jnp.int32),
            pltpu.VMEM((EB,), jnp.int32),
            pltpu.VMEM((EB, F), jnp.float32),
            pltpu.VMEM((EB, F), jnp.float32),
            pltpu.VMEM_SHARED((N, F), jnp.float32),
            pltpu.SemaphoreType.DMA,
        ],
    )
    def conv(x_lo_hbm, x_hi_hbm, ea_lo_hbm, ea_hi_hbm, src_hbm, dst_hbm,
             out_lo, out_hi, src_v, dst_v, rows_v, ea_v, aggr_sh, sem):
        c = lax.axis_index("c")
        s = lax.axis_index("s")

        zero16 = jnp.zeros((16,), jnp.float32)

        def zrow(i, _):
            for j in range(F // 16):
                rows_v[i, pl.ds(j * 16, 16)] = zero16
            return 0
        lax.fori_loop(0, EB, zrow, 0)

        row0 = s * ROWS_PER_TILE
        done = 0
        while done < ROWS_PER_TILE:
            cnt = min(EB, ROWS_PER_TILE - done)
            pltpu.sync_copy(rows_v.at[pl.ds(0, cnt)],
                            aggr_sh.at[pl.ds(row0 + done, cnt)])
            done += cnt
        plsc.subcore_barrier()

        ebase = s * EDGES_PER_TILE

        def edge_pass(x_hbm, ea_hbm):
            def step(k, _):
                e0 = ebase + k * EB
                pltpu.sync_copy(src_hbm.at[pl.ds(e0, EB)], src_v)
                pltpu.sync_copy(dst_hbm.at[pl.ds(e0, EB)], dst_v)
                gather = pltpu.async_copy(x_hbm.at[src_v], rows_v, sem)
                pltpu.sync_copy(ea_hbm.at[pl.ds(e0, EB)], ea_v)
                gather.wait()

                def row(i, _):
                    for j in range(F // 16):
                        sl = pl.ds(j * 16, 16)
                        rows_v[i, sl] = jnp.maximum(rows_v[i, sl] + ea_v[i, sl],
                                                    0.0)
                    return 0
                lax.fori_loop(0, EB, row, 0)

                pltpu.sync_copy(rows_v, aggr_sh.at[dst_v], add=True)
                return 0
            lax.fori_loop(0, STEPS, step, 0)

        @pl.when(c == 0)
        def _():
            edge_pass(x_lo_hbm, ea_lo_hbm)

        @pl.when(c == 1)
        def _():
            edge_pass(x_hi_hbm, ea_hi_hbm)

        plsc.subcore_barrier()

        @pl.when(c == 0)
        def _():
            pltpu.sync_copy(aggr_sh.at[pl.ds(row0, ROWS_PER_TILE)],
                            out_lo.at[pl.ds(row0, ROWS_PER_TILE)])

        @pl.when(c == 1)
        def _():
            pltpu.sync_copy(aggr_sh.at[pl.ds(row0, ROWS_PER_TILE)],
                            out_hi.at[pl.ds(row0, ROWS_PER_TILE)])

    return conv(x_lo, x_hi, ea_lo, ea_hi, src, dst)


def _edge_mlp(edge_attr, e_w1, e_b1, e_w2, e_b2):
    BE = 2000

    def body(ea_ref, w1_ref, b1_ref, w2_ref, b2_ref, lo_ref, hi_ref):
        h = jnp.maximum(
            jnp.dot(ea_ref[...], w1_ref[...], preferred_element_type=jnp.float32)
            + b1_ref[...], 0.0)
        o = jnp.dot(h, w2_ref[...], preferred_element_type=jnp.float32) + b2_ref[...]
        lo_ref[...] = o[:, :F]
        hi_ref[...] = o[:, F:]

    return pl.pallas_call(
        body,
        grid=(E // BE,),
        in_specs=[
            pl.BlockSpec((BE, ED), lambda i: (i, 0)),
            pl.BlockSpec((ED, H), lambda i: (0, 0)),
            pl.BlockSpec((1, H), lambda i: (0, 0)),
            pl.BlockSpec((H, H), lambda i: (0, 0)),
            pl.BlockSpec((1, H), lambda i: (0, 0)),
        ],
        out_specs=[
            pl.BlockSpec((BE, F), lambda i: (i, 0)),
            pl.BlockSpec((BE, F), lambda i: (i, 0)),
        ],
        out_shape=[jax.ShapeDtypeStruct((E, F), jnp.float32),
                   jax.ShapeDtypeStruct((E, F), jnp.float32)],
    )(edge_attr, e_w1, e_b1.reshape(1, H), e_w2, e_b2.reshape(1, H))


def _embed(z2, emb_pad):
    BN = 2000
    K = emb_pad.shape[0]

    def body(z_ref, emb_ref, lo_ref, hi_ref):
        z = z_ref[...]
        ids = lax.broadcasted_iota(jnp.int32, (BN, K), 1)
        onehot = jnp.where(ids == z, 1.0, 0.0).astype(jnp.float32)
        x = jnp.dot(onehot, emb_ref[...], preferred_element_type=jnp.float32)
        lo_ref[...] = x[:, :F]
        hi_ref[...] = x[:, F:]

    return pl.pallas_call(
        body,
        grid=(N // BN,),
        in_specs=[
            pl.BlockSpec((BN, 1), lambda i: (i, 0)),
            pl.BlockSpec((K, H), lambda i: (0, 0)),
        ],
        out_specs=[
            pl.BlockSpec((BN, F), lambda i: (i, 0)),
            pl.BlockSpec((BN, F), lambda i: (i, 0)),
        ],
        out_shape=[jax.ShapeDtypeStruct((N, F), jnp.float32),
                   jax.ShapeDtypeStruct((N, F), jnp.float32)],
    )(z2, emb_pad)


def _node_mlp(x_lo, x_hi, a_lo, a_hi, w1, b1, w2, b2, inner_relu):
    BN = 2000

    def body(xl_ref, xh_ref, al_ref, ah_ref, w1_ref, b1_ref, w2_ref, b2_ref,
             ol_ref, oh_ref):
        x = jnp.concatenate([xl_ref[...], xh_ref[...]], axis=1)
        a = jnp.concatenate([al_ref[...], ah_ref[...]], axis=1)
        h = x + a
        h = jnp.maximum(
            jnp.dot(h, w1_ref[...], preferred_element_type=jnp.float32)
            + b1_ref[...], 0.0)
        h = jnp.dot(h, w2_ref[...], preferred_element_type=jnp.float32) + b2_ref[...]
        if inner_relu:
            h = jnp.maximum(h, 0.0)
        h = h + x
        ol_ref[...] = h[:, :F]
        oh_ref[...] = h[:, F:]

    return pl.pallas_call(
        body,
        grid=(N // BN,),
        in_specs=[
            pl.BlockSpec((BN, F), lambda i: (i, 0)),
            pl.BlockSpec((BN, F), lambda i: (i, 0)),
            pl.BlockSpec((BN, F), lambda i: (i, 0)),
            pl.BlockSpec((BN, F), lambda i: (i, 0)),
            pl.BlockSpec((H, H), lambda i: (0, 0)),
            pl.BlockSpec((1, H), lambda i: (0, 0)),
            pl.BlockSpec((H, H), lambda i: (0, 0)),
            pl.BlockSpec((1, H), lambda i: (0, 0)),
        ],
        out_specs=[
            pl.BlockSpec((BN, F), lambda i: (i, 0)),
            pl.BlockSpec((BN, F), lambda i: (i, 0)),
        ],
        out_shape=[jax.ShapeDtypeStruct((N, F), jnp.float32),
                   jax.ShapeDtypeStruct((N, F), jnp.float32)],
    )(x_lo, x_hi, a_lo, a_hi, w1, b1.reshape(1, H), w2, b2.reshape(1, H))


def _pool_head(x_lo, x_hi, batch2, f_w1, f_b1, f_w2, f_b2):
    BN = 2000
    NBLK = N // BN

    def body(xl_ref, xh_ref, b_ref, w1_ref, b1_ref, w2_ref, b2_ref, out_ref,
             pooled_ref):
        blk = pl.program_id(0)

        @pl.when(blk == 0)
        def _():
            pooled_ref[...] = jnp.full((G, H), -jnp.inf, jnp.float32)

        x = jnp.concatenate([xl_ref[...], xh_ref[...]], axis=1)
        b = b_ref[...]

        def grp(g, _):
            m = jnp.where(b == g, x, -jnp.inf)
            mx = jnp.max(m, axis=0).reshape(1, H)
            cur = pooled_ref[pl.ds(g, 1), :]
            pooled_ref[pl.ds(g, 1), :] = jnp.maximum(cur, mx)
            return 0
        lax.fori_loop(0, G, grp, 0)

        @pl.when(blk == NBLK - 1)
        def _():
            p = pooled_ref[...]
            h = jnp.maximum(
                jnp.dot(p, w1_ref[...], preferred_element_type=jnp.float32)
                + b1_ref[...], 0.0)
            out_ref[...] = (
                jnp.dot(h, w2_ref[...], preferred_element_type=jnp.float32)
                + b2_ref[...])

    return pl.pallas_call(
        body,
        grid=(NBLK,),
        in_specs=[
            pl.BlockSpec((BN, F), lambda i: (i, 0)),
            pl.BlockSpec((BN, F), lambda i: (i, 0)),
            pl.BlockSpec((BN, 1), lambda i: (i, 0)),
            pl.BlockSpec((H, H), lambda i: (0, 0)),
            pl.BlockSpec((1, H), lambda i: (0, 0)),
            pl.BlockSpec((H, L), lambda i: (0, 0)),
            pl.BlockSpec((1, L), lambda i: (0, 0)),
        ],
        out_specs=pl.BlockSpec((G, L), lambda i: (0, 0)),
        out_shape=jax.ShapeDtypeStruct((G, L), jnp.float32),
        scratch_shapes=[pltpu.VMEM((G, H), jnp.float32)],
    )(x_lo, x_hi, batch2, f_w1, f_b1.reshape(1, H), f_w2, f_b2.reshape(1, L))


def kernel(atom_types, edge_index, edge_attr, batch, node_emb,
           e_w1, e_b1, e_w2, e_b2,
           c_w1, c_b1, c_w2, c_b2,
           f_w1, f_b1, f_w2, f_b2):
    src = edge_index[0].astype(jnp.int32)
    dst = edge_index[1].astype(jnp.int32)
    z2 = (atom_types.astype(jnp.int32) - 1).reshape(N, 1)
    batch2 = batch.astype(jnp.int32).reshape(N, 1)

    emb_pad = jnp.zeros((128, H), jnp.float32).at[:100].set(node_emb)

    ea_lo, ea_hi = _edge_mlp(edge_attr, e_w1, e_b1, e_w2, e_b2)
    x_lo, x_hi = _embed(z2, emb_pad)

    for i in range(NC):
        a_lo, a_hi = _sc_conv(x_lo, x_hi, ea_lo, ea_hi, src, dst)
        x_lo, x_hi = _node_mlp(x_lo, x_hi, a_lo, a_hi,
                               c_w1[i], c_b1[i], c_w2[i], c_b2[i],
                               inner_relu=(i < NC - 1))

    return _pool_head(x_lo, x_hi, batch2, f_w1, f_b1, f_w2, f_b2)


# speedup vs baseline: 2.3094x; 2.3094x over previous
import functools

import jax
import jax.numpy as jnp
from jax import lax
from jax.experimental import pallas as pl
from jax.experimental.pallas import tpu as pltpu
from jax.experimental.pallas import tpu_sc as plsc

N = 10000
E = 160000
H = 256
L = 128
ED = 16
NC = 3
G = 64

F = H // 2
SC_TILES = 16
EDGES_PER_TILE = E // SC_TILES
EB = 80
STEPS = EDGES_PER_TILE // EB
ROWS_PER_TILE = 624
ROWS_TAIL = N - SC_TILES * ROWS_PER_TILE


def _sc_conv(x_lo, x_hi, ea_lo, ea_hi, src, dst):
    mesh = plsc.VectorSubcoreMesh(core_axis_name="c", subcore_axis_name="s",
                                  num_cores=2, num_subcores=16)

    @functools.partial(
        pl.kernel,
        out_type=[jax.ShapeDtypeStruct((N, F), jnp.float32),
                  jax.ShapeDtypeStruct((N, F), jnp.float32)],
        mesh=mesh,
        scratch_types=[
            pltpu.VMEM((EB,), jnp.int32),
            pltpu.VMEM((EB,), jnp.int32),
            pltpu.VMEM((EB, F), jnp.float32),
            pltpu.VMEM((EB, F), jnp.float32),
            pltpu.VMEM_SHARED((N, F), jnp.float32),
            pltpu.SemaphoreType.DMA,
        ],
    )
    def conv(x_lo_hbm, x_hi_hbm, ea_lo_hbm, ea_hi_hbm, src_hbm, dst_hbm,
             out_lo, out_hi, src_v, dst_v, rows_v, ea_v, aggr_sh, sem):
        c = lax.axis_index("c")
        s = lax.axis_index("s")

        zero16 = jnp.zeros((16,), jnp.float32)

        def zrow(i, _):
            for j in range(F // 16):
                rows_v[i, pl.ds(j * 16, 16)] = zero16
            return 0
        lax.fori_loop(0, EB, zrow, 0)

        row0 = s * ROWS_PER_TILE
        done = 0
        while done < ROWS_PER_TILE:
            cnt = min(EB, ROWS_PER_TILE - done)
            pltpu.sync_copy(rows_v.at[pl.ds(0, cnt)],
                            aggr_sh.at[pl.ds(row0 + done, cnt)])
            done += cnt

        @pl.when(s == SC_TILES - 1)
        def _():
            pltpu.sync_copy(rows_v.at[pl.ds(0, ROWS_TAIL)],
                            aggr_sh.at[pl.ds(SC_TILES * ROWS_PER_TILE,
                                             ROWS_TAIL)])
        plsc.subcore_barrier()

        ebase = s * EDGES_PER_TILE

        def edge_pass(x_hbm, ea_hbm):
            def step(k, _):
                e0 = ebase + k * EB
                pltpu.sync_copy(src_hbm.at[pl.ds(e0, EB)], src_v)
                pltpu.sync_copy(dst_hbm.at[pl.ds(e0, EB)], dst_v)
                gather = pltpu.async_copy(x_hbm.at[src_v], rows_v, sem)
                pltpu.sync_copy(ea_hbm.at[pl.ds(e0, EB)], ea_v)
                gather.wait()

                def row(i, _):
                    for j in range(F // 16):
                        sl = pl.ds(j * 16, 16)
                        rows_v[i, sl] = jnp.maximum(rows_v[i, sl] + ea_v[i, sl],
                                                    0.0)
                    return 0
                lax.fori_loop(0, EB, row, 0)

                pltpu.sync_copy(rows_v, aggr_sh.at[dst_v], add=True)
                return 0
            lax.fori_loop(0, STEPS, step, 0)

        @pl.when(c == 0)
        def _():
            edge_pass(x_lo_hbm, ea_lo_hbm)

        @pl.when(c == 1)
        def _():
            edge_pass(x_hi_hbm, ea_hi_hbm)

        plsc.subcore_barrier()

        def copy_out(out_ref):
            pltpu.sync_copy(aggr_sh.at[pl.ds(row0, ROWS_PER_TILE)],
                            out_ref.at[pl.ds(row0, ROWS_PER_TILE)])

            @pl.when(s == SC_TILES - 1)
            def _():
                base = SC_TILES * ROWS_PER_TILE
                pltpu.sync_copy(aggr_sh.at[pl.ds(base, ROWS_TAIL)],
                                out_ref.at[pl.ds(base, ROWS_TAIL)])

        @pl.when(c == 0)
        def _():
            copy_out(out_lo)

        @pl.when(c == 1)
        def _():
            copy_out(out_hi)

    return conv(x_lo, x_hi, ea_lo, ea_hi, src, dst)


def _edge_mlp(edge_attr, e_w1, e_b1, e_w2, e_b2):
    BE = 2000

    def body(ea_ref, w1_ref, b1_ref, w2_ref, b2_ref, lo_ref, hi_ref):
        h = jnp.maximum(
            jnp.dot(ea_ref[...], w1_ref[...], preferred_element_type=jnp.float32)
            + b1_ref[...], 0.0)
        o = jnp.dot(h, w2_ref[...], preferred_element_type=jnp.float32) + b2_ref[...]
        lo_ref[...] = o[:, :F]
        hi_ref[...] = o[:, F:]

    return pl.pallas_call(
        body,
        grid=(E // BE,),
        in_specs=[
            pl.BlockSpec((BE, ED), lambda i: (i, 0)),
            pl.BlockSpec((ED, H), lambda i: (0, 0)),
            pl.BlockSpec((1, H), lambda i: (0, 0)),
            pl.BlockSpec((H, H), lambda i: (0, 0)),
            pl.BlockSpec((1, H), lambda i: (0, 0)),
        ],
        out_specs=[
            pl.BlockSpec((BE, F), lambda i: (i, 0)),
            pl.BlockSpec((BE, F), lambda i: (i, 0)),
        ],
        out_shape=[jax.ShapeDtypeStruct((E, F), jnp.float32),
                   jax.ShapeDtypeStruct((E, F), jnp.float32)],
    )(edge_attr, e_w1, e_b1.reshape(1, H), e_w2, e_b2.reshape(1, H))


def _embed(z2, emb_pad):
    BN = 2000
    K = emb_pad.shape[0]

    def body(z_ref, emb_ref, lo_ref, hi_ref):
        z = z_ref[...]
        ids = lax.broadcasted_iota(jnp.int32, (BN, K), 1)
        onehot = jnp.where(ids == z, 1.0, 0.0).astype(jnp.float32)
        x = jnp.dot(onehot, emb_ref[...], preferred_element_type=jnp.float32)
        lo_ref[...] = x[:, :F]
        hi_ref[...] = x[:, F:]

    return pl.pallas_call(
        body,
        grid=(N // BN,),
        in_specs=[
            pl.BlockSpec((BN, 1), lambda i: (i, 0)),
            pl.BlockSpec((K, H), lambda i: (0, 0)),
        ],
        out_specs=[
            pl.BlockSpec((BN, F), lambda i: (i, 0)),
            pl.BlockSpec((BN, F), lambda i: (i, 0)),
        ],
        out_shape=[jax.ShapeDtypeStruct((N, F), jnp.float32),
                   jax.ShapeDtypeStruct((N, F), jnp.float32)],
    )(z2, emb_pad)


def _node_mlp(x_lo, x_hi, a_lo, a_hi, w1, b1, w2, b2, inner_relu):
    BN = 2000

    def body(xl_ref, xh_ref, al_ref, ah_ref, w1_ref, b1_ref, w2_ref, b2_ref,
             ol_ref, oh_ref):
        x = jnp.concatenate([xl_ref[...], xh_ref[...]], axis=1)
        a = jnp.concatenate([al_ref[...], ah_ref[...]], axis=1)
        h = x + a
        h = jnp.maximum(
            jnp.dot(h, w1_ref[...], preferred_element_type=jnp.float32)
            + b1_ref[...], 0.0)
        h = jnp.dot(h, w2_ref[...], preferred_element_type=jnp.float32) + b2_ref[...]
        if inner_relu:
            h = jnp.maximum(h, 0.0)
        h = h + x
        ol_ref[...] = h[:, :F]
        oh_ref[...] = h[:, F:]

    return pl.pallas_call(
        body,
        grid=(N // BN,),
        in_specs=[
            pl.BlockSpec((BN, F), lambda i: (i, 0)),
            pl.BlockSpec((BN, F), lambda i: (i, 0)),
            pl.BlockSpec((BN, F), lambda i: (i, 0)),
            pl.BlockSpec((BN, F), lambda i: (i, 0)),
            pl.BlockSpec((H, H), lambda i: (0, 0)),
            pl.BlockSpec((1, H), lambda i: (0, 0)),
            pl.BlockSpec((H, H), lambda i: (0, 0)),
            pl.BlockSpec((1, H), lambda i: (0, 0)),
        ],
        out_specs=[
            pl.BlockSpec((BN, F), lambda i: (i, 0)),
            pl.BlockSpec((BN, F), lambda i: (i, 0)),
        ],
        out_shape=[jax.ShapeDtypeStruct((N, F), jnp.float32),
                   jax.ShapeDtypeStruct((N, F), jnp.float32)],
    )(x_lo, x_hi, a_lo, a_hi, w1, b1.reshape(1, H), w2, b2.reshape(1, H))


def _pool_head(x_lo, x_hi, batch2, f_w1, f_b1, f_w2, f_b2):
    BN = 2000
    NBLK = N // BN

    def body(xl_ref, xh_ref, b_ref, w1_ref, b1_ref, w2_ref, b2_ref, out_ref,
             pooled_ref):
        blk = pl.program_id(0)

        @pl.when(blk == 0)
        def _():
            pooled_ref[...] = jnp.full((G, H), -jnp.inf, jnp.float32)

        x = jnp.concatenate([xl_ref[...], xh_ref[...]], axis=1)
        b = b_ref[...]

        def grp(g, _):
            m = jnp.where(b == g, x, -jnp.inf)
            mx = jnp.max(m, axis=0).reshape(1, H)
            cur = pooled_ref[pl.ds(g, 1), :]
            pooled_ref[pl.ds(g, 1), :] = jnp.maximum(cur, mx)
            return 0
        lax.fori_loop(0, G, grp, 0)

        @pl.when(blk == NBLK - 1)
        def _():
            p = pooled_ref[...]
            h = jnp.maximum(
                jnp.dot(p, w1_ref[...], preferred_element_type=jnp.float32)
                + b1_ref[...], 0.0)
            out_ref[...] = (
                jnp.dot(h, w2_ref[...], preferred_element_type=jnp.float32)
                + b2_ref[...])

    return pl.pallas_call(
        body,
        grid=(NBLK,),
        in_specs=[
            pl.BlockSpec((BN, F), lambda i: (i, 0)),
            pl.BlockSpec((BN, F), lambda i: (i, 0)),
            pl.BlockSpec((BN, 1), lambda i: (i, 0)),
            pl.BlockSpec((H, H), lambda i: (0, 0)),
            pl.BlockSpec((1, H), lambda i: (0, 0)),
            pl.BlockSpec((H, L), lambda i: (0, 0)),
            pl.BlockSpec((1, L), lambda i: (0, 0)),
        ],
        out_specs=pl.BlockSpec((G, L), lambda i: (0, 0)),
        out_shape=jax.ShapeDtypeStruct((G, L), jnp.float32),
        scratch_shapes=[pltpu.VMEM((G, H), jnp.float32)],
    )(x_lo, x_hi, batch2, f_w1, f_b1.reshape(1, H), f_w2, f_b2.reshape(1, L))


def kernel(atom_types, edge_index, edge_attr, batch, node_emb,
           e_w1, e_b1, e_w2, e_b2,
           c_w1, c_b1, c_w2, c_b2,
           f_w1, f_b1, f_w2, f_b2):
    src = edge_index[0].astype(jnp.int32)
    dst = edge_index[1].astype(jnp.int32)
    z2 = (atom_types.astype(jnp.int32) - 1).reshape(N, 1)
    batch2 = batch.astype(jnp.int32).reshape(N, 1)

    emb_pad = jnp.zeros((128, H), jnp.float32).at[:100].set(node_emb)

    ea_lo, ea_hi = _edge_mlp(edge_attr, e_w1, e_b1, e_w2, e_b2)
    x_lo, x_hi = _embed(z2, emb_pad)

    for i in range(NC):
        a_lo, a_hi = _sc_conv(x_lo, x_hi, ea_lo, ea_hi, src, dst)
        x_lo, x_hi = _node_mlp(x_lo, x_hi, a_lo, a_hi,
                               c_w1[i], c_b1[i], c_w2[i], c_b2[i],
                               inner_relu=(i < NC - 1))

    return _pool_head(x_lo, x_hi, batch2, f_w1, f_b1, f_w2, f_b2)


# speedup vs baseline: 3.8909x; 1.6848x over previous
import functools

import jax
import jax.numpy as jnp
from jax import lax
from jax.experimental import pallas as pl
from jax.experimental.pallas import tpu as pltpu
from jax.experimental.pallas import tpu_sc as plsc

N = 10000
E = 160000
H = 256
L = 128
ED = 16
NC = 3
G = 64

F = H // 2
SC_TILES = 16
EDGES_PER_TILE = E // SC_TILES
EB = 40
STEPS = EDGES_PER_TILE // EB
ROWS_PER_TILE = 624
ROWS_TAIL = N - SC_TILES * ROWS_PER_TILE


NBUF = 3
MAIN_STEPS = (STEPS // NBUF) * NBUF


def _sc_conv(x_lo, x_hi, ea_lo, ea_hi, src3, dst3):
    mesh = plsc.VectorSubcoreMesh(core_axis_name="c", subcore_axis_name="s",
                                  num_cores=2, num_subcores=16)

    @functools.partial(
        pl.kernel,
        out_type=[jax.ShapeDtypeStruct((N, F), jnp.float32),
                  jax.ShapeDtypeStruct((N, F), jnp.float32)],
        mesh=mesh,
        scratch_types=[
            [pltpu.VMEM((EB,), jnp.int32)] * NBUF,
            [pltpu.VMEM((EB,), jnp.int32)] * NBUF,
            [pltpu.VMEM((EB, F), jnp.float32)] * NBUF,
            [pltpu.VMEM((EB, F), jnp.float32)] * NBUF,
            pltpu.VMEM_SHARED((N, F), jnp.float32),
            [pltpu.SemaphoreType.DMA] * NBUF,
            [pltpu.SemaphoreType.DMA] * NBUF,
            [pltpu.SemaphoreType.DMA] * NBUF,
        ],
    )
    def conv(x_lo_hbm, x_hi_hbm, ea_lo_hbm, ea_hi_hbm, src_hbm, dst_hbm,
             out_lo, out_hi, src_ix, dst_ix, rows, eav, aggr_sh,
             sem_io, sem_sc, sem_six):
        c = lax.axis_index("c")
        s = lax.axis_index("s")

        zero16 = jnp.zeros((16,), jnp.float32)

        @plsc.parallel_loop(0, EB)
        def _(i):
            for j in range(F // 16):
                rows[0][i, pl.ds(j * 16, 16)] = zero16

        row0 = s * ROWS_PER_TILE
        done = 0
        while done < ROWS_PER_TILE:
            cnt = min(EB, ROWS_PER_TILE - done)
            pltpu.sync_copy(rows[0].at[pl.ds(0, cnt)],
                            aggr_sh.at[pl.ds(row0 + done, cnt)])
            done += cnt

        @pl.when(s == SC_TILES - 1)
        def _():
            pltpu.sync_copy(rows[0].at[pl.ds(0, ROWS_TAIL)],
                            aggr_sh.at[pl.ds(SC_TILES * ROWS_PER_TILE,
                                             ROWS_TAIL)])
        plsc.subcore_barrier()

        ebase = s * EDGES_PER_TILE

        def edge_pass(x_hbm, ea_hbm):
            def load_src(b, k):
                pltpu.async_copy(src_hbm.at[s, k], src_ix[b], sem_six[b])

            def wait_six(b):
                pltpu.make_async_copy(src_hbm.at[s, 0], src_ix[b],
                                      sem_six[b]).wait()

            def issue_io(b, k):
                pltpu.async_copy(x_hbm.at[src_ix[b]], rows[b], sem_io[b])
                pltpu.async_copy(ea_hbm.at[pl.ds(ebase + k * EB, EB)],
                                 eav[b], sem_io[b])
                pltpu.async_copy(dst_hbm.at[s, k], dst_ix[b], sem_io[b])

            def wait_io(b):
                pltpu.make_async_copy(ea_hbm.at[pl.ds(0, EB)], rows[b],
                                      sem_io[b]).wait()
                pltpu.make_async_copy(ea_hbm.at[pl.ds(0, EB)], eav[b],
                                      sem_io[b]).wait()
                pltpu.make_async_copy(src_hbm.at[s, 0], dst_ix[b],
                                      sem_io[b]).wait()

            def compute(b):
                @plsc.parallel_loop(0, EB, unroll=2)
                def _(i):
                    for j in range(F // 16):
                        sl = pl.ds(j * 16, 16)
                        rows[b][i, sl] = jnp.maximum(
                            rows[b][i, sl] + eav[b][i, sl], 0.0)

            def issue_scatter(b):
                pltpu.async_copy(rows[b], aggr_sh.at[dst_ix[b]],
                                 sem_sc[b], add=True)

            def wait_scatter(b):
                pltpu.make_async_copy(ea_hbm.at[pl.ds(0, EB)], rows[b],
                                      sem_sc[b]).wait()

            def step_fn(k, p, q):
                wait_io(p)

                @pl.when(k + NBUF < STEPS)
                def _():
                    load_src(p, k + NBUF)

                compute(p)
                issue_scatter(p)

                @pl.when((k >= 1) & (k + 2 < STEPS))
                def _():
                    wait_scatter(q)
                    wait_six(q)
                    issue_io(q, k + 2)

            for b in range(NBUF):
                pltpu.sync_copy(src_hbm.at[s, b], src_ix[b])
            for b in range(NBUF):
                issue_io(b, b)

            def tri(i, _):
                k0 = i * NBUF
                for b in range(NBUF):
                    step_fn(k0 + b, b, (b - 1) % NBUF)
                return 0
            lax.fori_loop(0, STEPS // NBUF, tri, 0)

            for k in range(MAIN_STEPS, STEPS):
                p = k % NBUF
                wait_io(p)
                compute(p)
                issue_scatter(p)

            for b in range(NBUF):
                wait_scatter(b)

        @pl.when(c == 0)
        def _():
            edge_pass(x_lo_hbm, ea_lo_hbm)

        @pl.when(c == 1)
        def _():
            edge_pass(x_hi_hbm, ea_hi_hbm)

        plsc.subcore_barrier()

        def copy_out(out_ref):
            pltpu.sync_copy(aggr_sh.at[pl.ds(row0, ROWS_PER_TILE)],
                            out_ref.at[pl.ds(row0, ROWS_PER_TILE)])

            @pl.when(s == SC_TILES - 1)
            def _():
                base = SC_TILES * ROWS_PER_TILE
                pltpu.sync_copy(aggr_sh.at[pl.ds(base, ROWS_TAIL)],
                                out_ref.at[pl.ds(base, ROWS_TAIL)])

        @pl.when(c == 0)
        def _():
            copy_out(out_lo)

        @pl.when(c == 1)
        def _():
            copy_out(out_hi)

    return conv(x_lo, x_hi, ea_lo, ea_hi, src3, dst3)


def _edge_mlp(edge_attr, e_w1, e_b1, e_w2, e_b2):
    BE = 2000

    def body(ea_ref, w1_ref, b1_ref, w2_ref, b2_ref, lo_ref, hi_ref):
        h = jnp.maximum(
            jnp.dot(ea_ref[...], w1_ref[...], preferred_element_type=jnp.float32)
            + b1_ref[...], 0.0)
        o = jnp.dot(h, w2_ref[...], preferred_element_type=jnp.float32) + b2_ref[...]
        lo_ref[...] = o[:, :F]
        hi_ref[...] = o[:, F:]

    return pl.pallas_call(
        body,
        grid=(E // BE,),
        in_specs=[
            pl.BlockSpec((BE, ED), lambda i: (i, 0)),
            pl.BlockSpec((ED, H), lambda i: (0, 0)),
            pl.BlockSpec((1, H), lambda i: (0, 0)),
            pl.BlockSpec((H, H), lambda i: (0, 0)),
            pl.BlockSpec((1, H), lambda i: (0, 0)),
        ],
        out_specs=[
            pl.BlockSpec((BE, F), lambda i: (i, 0)),
            pl.BlockSpec((BE, F), lambda i: (i, 0)),
        ],
        out_shape=[jax.ShapeDtypeStruct((E, F), jnp.float32),
                   jax.ShapeDtypeStruct((E, F), jnp.float32)],
    )(edge_attr, e_w1, e_b1.reshape(1, H), e_w2, e_b2.reshape(1, H))


def _embed(z2, emb_pad):
    BN = 2000
    K = emb_pad.shape[0]

    def body(z_ref, emb_ref, lo_ref, hi_ref):
        z = z_ref[...]
        ids = lax.broadcasted_iota(jnp.int32, (BN, K), 1)
        onehot = jnp.where(ids == z, 1.0, 0.0).astype(jnp.float32)
        x = jnp.dot(onehot, emb_ref[...], preferred_element_type=jnp.float32)
        lo_ref[...] = x[:, :F]
        hi_ref[...] = x[:, F:]

    return pl.pallas_call(
        body,
        grid=(N // BN,),
        in_specs=[
            pl.BlockSpec((BN, 1), lambda i: (i, 0)),
            pl.BlockSpec((K, H), lambda i: (0, 0)),
        ],
        out_specs=[
            pl.BlockSpec((BN, F), lambda i: (i, 0)),
            pl.BlockSpec((BN, F), lambda i: (i, 0)),
        ],
        out_shape=[jax.ShapeDtypeStruct((N, F), jnp.float32),
                   jax.ShapeDtypeStruct((N, F), jnp.float32)],
    )(z2, emb_pad)


def _node_mlp(x_lo, x_hi, a_lo, a_hi, w1, b1, w2, b2, inner_relu):
    BN = 2000

    def body(xl_ref, xh_ref, al_ref, ah_ref, w1_ref, b1_ref, w2_ref, b2_ref,
             ol_ref, oh_ref):
        x = jnp.concatenate([xl_ref[...], xh_ref[...]], axis=1)
        a = jnp.concatenate([al_ref[...], ah_ref[...]], axis=1)
        h = x + a
        h = jnp.maximum(
            jnp.dot(h, w1_ref[...], preferred_element_type=jnp.float32)
            + b1_ref[...], 0.0)
        h = jnp.dot(h, w2_ref[...], preferred_element_type=jnp.float32) + b2_ref[...]
        if inner_relu:
            h = jnp.maximum(h, 0.0)
        h = h + x
        ol_ref[...] = h[:, :F]
        oh_ref[...] = h[:, F:]

    return pl.pallas_call(
        body,
        grid=(N // BN,),
        in_specs=[
            pl.BlockSpec((BN, F), lambda i: (i, 0)),
            pl.BlockSpec((BN, F), lambda i: (i, 0)),
            pl.BlockSpec((BN, F), lambda i: (i, 0)),
            pl.BlockSpec((BN, F), lambda i: (i, 0)),
            pl.BlockSpec((H, H), lambda i: (0, 0)),
            pl.BlockSpec((1, H), lambda i: (0, 0)),
            pl.BlockSpec((H, H), lambda i: (0, 0)),
            pl.BlockSpec((1, H), lambda i: (0, 0)),
        ],
        out_specs=[
            pl.BlockSpec((BN, F), lambda i: (i, 0)),
            pl.BlockSpec((BN, F), lambda i: (i, 0)),
        ],
        out_shape=[jax.ShapeDtypeStruct((N, F), jnp.float32),
                   jax.ShapeDtypeStruct((N, F), jnp.float32)],
    )(x_lo, x_hi, a_lo, a_hi, w1, b1.reshape(1, H), w2, b2.reshape(1, H))


def _pool_head(x_lo, x_hi, batch2, f_w1, f_b1, f_w2, f_b2):
    BN = 2000
    NBLK = N // BN

    def body(xl_ref, xh_ref, b_ref, w1_ref, b1_ref, w2_ref, b2_ref, out_ref,
             pooled_ref):
        blk = pl.program_id(0)

        @pl.when(blk == 0)
        def _():
            pooled_ref[...] = jnp.full((G, H), -jnp.inf, jnp.float32)

        x = jnp.concatenate([xl_ref[...], xh_ref[...]], axis=1)
        b = b_ref[...]

        def grp(g, _):
            m = jnp.where(b == g, x, -jnp.inf)
            mx = jnp.max(m, axis=0).reshape(1, H)
            cur = pooled_ref[pl.ds(g, 1), :]
            pooled_ref[pl.ds(g, 1), :] = jnp.maximum(cur, mx)
            return 0
        lax.fori_loop(0, G, grp, 0)

        @pl.when(blk == NBLK - 1)
        def _():
            p = pooled_ref[...]
            h = jnp.maximum(
                jnp.dot(p, w1_ref[...], preferred_element_type=jnp.float32)
                + b1_ref[...], 0.0)
            out_ref[...] = (
                jnp.dot(h, w2_ref[...], preferred_element_type=jnp.float32)
                + b2_ref[...])

    return pl.pallas_call(
        body,
        grid=(NBLK,),
        in_specs=[
            pl.BlockSpec((BN, F), lambda i: (i, 0)),
            pl.BlockSpec((BN, F), lambda i: (i, 0)),
            pl.BlockSpec((BN, 1), lambda i: (i, 0)),
            pl.BlockSpec((H, H), lambda i: (0, 0)),
            pl.BlockSpec((1, H), lambda i: (0, 0)),
            pl.BlockSpec((H, L), lambda i: (0, 0)),
            pl.BlockSpec((1, L), lambda i: (0, 0)),
        ],
        out_specs=pl.BlockSpec((G, L), lambda i: (0, 0)),
        out_shape=jax.ShapeDtypeStruct((G, L), jnp.float32),
        scratch_shapes=[pltpu.VMEM((G, H), jnp.float32)],
    )(x_lo, x_hi, batch2, f_w1, f_b1.reshape(1, H), f_w2, f_b2.reshape(1, L))


def kernel(atom_types, edge_index, edge_attr, batch, node_emb,
           e_w1, e_b1, e_w2, e_b2,
           c_w1, c_b1, c_w2, c_b2,
           f_w1, f_b1, f_w2, f_b2):
    src3 = edge_index[0].astype(jnp.int32).reshape(SC_TILES, STEPS, EB)
    dst3 = edge_index[1].astype(jnp.int32).reshape(SC_TILES, STEPS, EB)
    z2 = (atom_types.astype(jnp.int32) - 1).reshape(N, 1)
    batch2 = batch.astype(jnp.int32).reshape(N, 1)

    emb_pad = jnp.zeros((128, H), jnp.float32).at[:100].set(node_emb)

    ea_lo, ea_hi = _edge_mlp(edge_attr, e_w1, e_b1, e_w2, e_b2)
    x_lo, x_hi = _embed(z2, emb_pad)

    for i in range(NC):
        a_lo, a_hi = _sc_conv(x_lo, x_hi, ea_lo, ea_hi, src3, dst3)
        x_lo, x_hi = _node_mlp(x_lo, x_hi, a_lo, a_hi,
                               c_w1[i], c_b1[i], c_w2[i], c_b2[i],
                               inner_relu=(i < NC - 1))

    return _pool_head(x_lo, x_hi, batch2, f_w1, f_b1, f_w2, f_b2)


# speedup vs baseline: 3.8991x; 1.0021x over previous
import functools

import jax
import jax.numpy as jnp
from jax import lax
from jax.experimental import pallas as pl
from jax.experimental.pallas import tpu as pltpu
from jax.experimental.pallas import tpu_sc as plsc

N = 10000
E = 160000
H = 256
L = 128
ED = 16
NC = 3
G = 64

F = H // 2
SC_TILES = 16
EDGES_PER_TILE = E // SC_TILES
EB = 40
STEPS = EDGES_PER_TILE // EB
ROWS_PER_TILE = 624
ROWS_TAIL = N - SC_TILES * ROWS_PER_TILE


NBUF = 3
MAIN_STEPS = (STEPS // NBUF) * NBUF


def _sc_conv(x_lo, x_hi, ea_lo, ea_hi, src3, dst3):
    mesh = plsc.VectorSubcoreMesh(core_axis_name="c", subcore_axis_name="s",
                                  num_cores=2, num_subcores=16)

    @functools.partial(
        pl.kernel,
        out_type=[jax.ShapeDtypeStruct((N, F), jnp.float32),
                  jax.ShapeDtypeStruct((N, F), jnp.float32)],
        mesh=mesh,
        scratch_types=[
            [pltpu.VMEM((EB,), jnp.int32)] * NBUF,
            [pltpu.VMEM((EB,), jnp.int32)] * NBUF,
            [pltpu.VMEM((EB, F), jnp.float32)] * NBUF,
            [pltpu.VMEM((EB, F), jnp.float32)] * NBUF,
            pltpu.VMEM_SHARED((N, F), jnp.float32),
            [pltpu.SemaphoreType.DMA] * NBUF,
            [pltpu.SemaphoreType.DMA] * NBUF,
            [pltpu.SemaphoreType.DMA] * NBUF,
        ],
    )
    def conv(x_lo_hbm, x_hi_hbm, ea_lo_hbm, ea_hi_hbm, src_hbm, dst_hbm,
             out_lo, out_hi, src_ix, dst_ix, rows, eav, aggr_sh,
             sem_io, sem_sc, sem_six):
        c = lax.axis_index("c")
        s = lax.axis_index("s")

        zero16 = jnp.zeros((16,), jnp.float32)

        @plsc.parallel_loop(0, EB)
        def _(i):
            for j in range(F // 16):
                rows[0][i, pl.ds(j * 16, 16)] = zero16

        row0 = s * ROWS_PER_TILE
        done = 0
        while done < ROWS_PER_TILE:
            cnt = min(EB, ROWS_PER_TILE - done)
            pltpu.sync_copy(rows[0].at[pl.ds(0, cnt)],
                            aggr_sh.at[pl.ds(row0 + done, cnt)])
            done += cnt

        @pl.when(s == SC_TILES - 1)
        def _():
            pltpu.sync_copy(rows[0].at[pl.ds(0, ROWS_TAIL)],
                            aggr_sh.at[pl.ds(SC_TILES * ROWS_PER_TILE,
                                             ROWS_TAIL)])
        plsc.subcore_barrier()

        ebase = s * EDGES_PER_TILE

        def edge_pass(x_hbm, ea_hbm):
            def load_src(b, k):
                pltpu.async_copy(src_hbm.at[s, k], src_ix[b], sem_six[b])

            def wait_six(b):
                pltpu.make_async_copy(src_hbm.at[s, 0], src_ix[b],
                                      sem_six[b]).wait()

            def issue_io(b, k):
                pltpu.async_copy(x_hbm.at[src_ix[b]], rows[b], sem_io[b])
                pltpu.async_copy(ea_hbm.at[pl.ds(ebase + k * EB, EB)],
                                 eav[b], sem_io[b])
                pltpu.async_copy(dst_hbm.at[s, k], dst_ix[b], sem_io[b])

            def wait_io(b):
                pltpu.make_async_copy(ea_hbm.at[pl.ds(0, EB)], rows[b],
                                      sem_io[b]).wait()
                pltpu.make_async_copy(ea_hbm.at[pl.ds(0, EB)], eav[b],
                                      sem_io[b]).wait()
                pltpu.make_async_copy(src_hbm.at[s, 0], dst_ix[b],
                                      sem_io[b]).wait()

            def compute(b):
                @plsc.parallel_loop(0, EB, unroll=4)
                def _(i):
                    for j in range(F // 16):
                        sl = pl.ds(j * 16, 16)
                        rows[b][i, sl] = jnp.maximum(
                            rows[b][i, sl] + eav[b][i, sl], 0.0)

            def issue_scatter(b):
                pltpu.async_copy(rows[b], aggr_sh.at[dst_ix[b]],
                                 sem_sc[b], add=True)

            def wait_scatter(b):
                pltpu.make_async_copy(ea_hbm.at[pl.ds(0, EB)], rows[b],
                                      sem_sc[b]).wait()

            def step_fn(k, p, q):
                wait_io(p)

                @pl.when(k + NBUF < STEPS)
                def _():
                    load_src(p, k + NBUF)

                compute(p)
                issue_scatter(p)

                @pl.when((k >= 1) & (k + 2 < STEPS))
                def _():
                    wait_scatter(q)
                    wait_six(q)
                    issue_io(q, k + 2)

            for b in range(NBUF):
                pltpu.sync_copy(src_hbm.at[s, b], src_ix[b])
            for b in range(NBUF):
                issue_io(b, b)

            def tri(i, _):
                k0 = i * NBUF
                for b in range(NBUF):
                    step_fn(k0 + b, b, (b - 1) % NBUF)
                return 0
            lax.fori_loop(0, STEPS // NBUF, tri, 0)

            for k in range(MAIN_STEPS, STEPS):
                p = k % NBUF
                wait_io(p)
                compute(p)
                issue_scatter(p)

            for b in range(NBUF):
                wait_scatter(b)

        @pl.when(c == 0)
        def _():
            edge_pass(x_lo_hbm, ea_lo_hbm)

        @pl.when(c == 1)
        def _():
            edge_pass(x_hi_hbm, ea_hi_hbm)

        plsc.subcore_barrier()

        def copy_out(out_ref):
            pltpu.sync_copy(aggr_sh.at[pl.ds(row0, ROWS_PER_TILE)],
                            out_ref.at[pl.ds(row0, ROWS_PER_TILE)])

            @pl.when(s == SC_TILES - 1)
            def _():
                base = SC_TILES * ROWS_PER_TILE
                pltpu.sync_copy(aggr_sh.at[pl.ds(base, ROWS_TAIL)],
                                out_ref.at[pl.ds(base, ROWS_TAIL)])

        @pl.when(c == 0)
        def _():
            copy_out(out_lo)

        @pl.when(c == 1)
        def _():
            copy_out(out_hi)

    return conv(x_lo, x_hi, ea_lo, ea_hi, src3, dst3)


def _edge_mlp(edge_attr, e_w1, e_b1, e_w2, e_b2):
    BE = 2000

    def body(ea_ref, w1_ref, b1_ref, w2_ref, b2_ref, lo_ref, hi_ref):
        h = jnp.maximum(
            jnp.dot(ea_ref[...], w1_ref[...], preferred_element_type=jnp.float32)
            + b1_ref[...], 0.0)
        o = jnp.dot(h.astype(jnp.bfloat16), w2_ref[...],
                    preferred_element_type=jnp.float32) + b2_ref[...]
        lo_ref[...] = o[:, :F]
        hi_ref[...] = o[:, F:]

    return pl.pallas_call(
        body,
        grid=(E // BE,),
        in_specs=[
            pl.BlockSpec((BE, ED), lambda i: (i, 0)),
            pl.BlockSpec((ED, H), lambda i: (0, 0)),
            pl.BlockSpec((1, H), lambda i: (0, 0)),
            pl.BlockSpec((H, H), lambda i: (0, 0)),
            pl.BlockSpec((1, H), lambda i: (0, 0)),
        ],
        out_specs=[
            pl.BlockSpec((BE, F), lambda i: (i, 0)),
            pl.BlockSpec((BE, F), lambda i: (i, 0)),
        ],
        out_shape=[jax.ShapeDtypeStruct((E, F), jnp.float32),
                   jax.ShapeDtypeStruct((E, F), jnp.float32)],
    )(edge_attr, e_w1, e_b1.reshape(1, H), e_w2.astype(jnp.bfloat16),
      e_b2.reshape(1, H))


def _embed(z2, emb_pad):
    BN = 2000
    K = emb_pad.shape[0]

    def body(z_ref, emb_ref, lo_ref, hi_ref):
        z = z_ref[...]
        ids = lax.broadcasted_iota(jnp.int32, (BN, K), 1)
        onehot = jnp.where(ids == z, 1.0, 0.0).astype(jnp.float32)
        x = jnp.dot(onehot, emb_ref[...], preferred_element_type=jnp.float32)
        lo_ref[...] = x[:, :F]
        hi_ref[...] = x[:, F:]

    return pl.pallas_call(
        body,
        grid=(N // BN,),
        in_specs=[
            pl.BlockSpec((BN, 1), lambda i: (i, 0)),
            pl.BlockSpec((K, H), lambda i: (0, 0)),
        ],
        out_specs=[
            pl.BlockSpec((BN, F), lambda i: (i, 0)),
            pl.BlockSpec((BN, F), lambda i: (i, 0)),
        ],
        out_shape=[jax.ShapeDtypeStruct((N, F), jnp.float32),
                   jax.ShapeDtypeStruct((N, F), jnp.float32)],
    )(z2, emb_pad)


def _node_mlp(x_lo, x_hi, a_lo, a_hi, w1, b1, w2, b2, inner_relu):
    BN = 2000

    def body(xl_ref, xh_ref, al_ref, ah_ref, w1_ref, b1_ref, w2_ref, b2_ref,
             ol_ref, oh_ref):
        x = jnp.concatenate([xl_ref[...], xh_ref[...]], axis=1)
        a = jnp.concatenate([al_ref[...], ah_ref[...]], axis=1)
        h = x + a
        h = jnp.maximum(
            jnp.dot(h, w1_ref[...], preferred_element_type=jnp.float32)
            + b1_ref[...], 0.0)
        h = jnp.dot(h, w2_ref[...], preferred_element_type=jnp.float32) + b2_ref[...]
        if inner_relu:
            h = jnp.maximum(h, 0.0)
        h = h + x
        ol_ref[...] = h[:, :F]
        oh_ref[...] = h[:, F:]

    return pl.pallas_call(
        body,
        grid=(N // BN,),
        in_specs=[
            pl.BlockSpec((BN, F), lambda i: (i, 0)),
            pl.BlockSpec((BN, F), lambda i: (i, 0)),
            pl.BlockSpec((BN, F), lambda i: (i, 0)),
            pl.BlockSpec((BN, F), lambda i: (i, 0)),
            pl.BlockSpec((H, H), lambda i: (0, 0)),
            pl.BlockSpec((1, H), lambda i: (0, 0)),
            pl.BlockSpec((H, H), lambda i: (0, 0)),
            pl.BlockSpec((1, H), lambda i: (0, 0)),
        ],
        out_specs=[
            pl.BlockSpec((BN, F), lambda i: (i, 0)),
            pl.BlockSpec((BN, F), lambda i: (i, 0)),
        ],
        out_shape=[jax.ShapeDtypeStruct((N, F), jnp.float32),
                   jax.ShapeDtypeStruct((N, F), jnp.float32)],
    )(x_lo, x_hi, a_lo, a_hi, w1, b1.reshape(1, H), w2, b2.reshape(1, H))


def _pool_head(x_lo, x_hi, batch2, f_w1, f_b1, f_w2, f_b2):
    BN = 2000
    NBLK = N // BN

    def body(xl_ref, xh_ref, b_ref, w1_ref, b1_ref, w2_ref, b2_ref, out_ref,
             pooled_ref):
        blk = pl.program_id(0)

        @pl.when(blk == 0)
        def _():
            pooled_ref[...] = jnp.full((G, H), -jnp.inf, jnp.float32)

        x = jnp.concatenate([xl_ref[...], xh_ref[...]], axis=1)
        b = b_ref[...]

        def grp(g, _):
            m = jnp.where(b == g, x, -jnp.inf)
            mx = jnp.max(m, axis=0).reshape(1, H)
            cur = pooled_ref[pl.ds(g, 1), :]
            pooled_ref[pl.ds(g, 1), :] = jnp.maximum(cur, mx)
            return 0
        lax.fori_loop(0, G, grp, 0)

        @pl.when(blk == NBLK - 1)
        def _():
            p = pooled_ref[...]
            h = jnp.maximum(
                jnp.dot(p, w1_ref[...], preferred_element_type=jnp.float32)
                + b1_ref[...], 0.0)
            out_ref[...] = (
                jnp.dot(h, w2_ref[...], preferred_element_type=jnp.float32)
                + b2_ref[...])

    return pl.pallas_call(
        body,
        grid=(NBLK,),
        in_specs=[
            pl.BlockSpec((BN, F), lambda i: (i, 0)),
            pl.BlockSpec((BN, F), lambda i: (i, 0)),
            pl.BlockSpec((BN, 1), lambda i: (i, 0)),
            pl.BlockSpec((H, H), lambda i: (0, 0)),
            pl.BlockSpec((1, H), lambda i: (0, 0)),
            pl.BlockSpec((H, L), lambda i: (0, 0)),
            pl.BlockSpec((1, L), lambda i: (0, 0)),
        ],
        out_specs=pl.BlockSpec((G, L), lambda i: (0, 0)),
        out_shape=jax.ShapeDtypeStruct((G, L), jnp.float32),
        scratch_shapes=[pltpu.VMEM((G, H), jnp.float32)],
    )(x_lo, x_hi, batch2, f_w1, f_b1.reshape(1, H), f_w2, f_b2.reshape(1, L))


def kernel(atom_types, edge_index, edge_attr, batch, node_emb,
           e_w1, e_b1, e_w2, e_b2,
           c_w1, c_b1, c_w2, c_b2,
           f_w1, f_b1, f_w2, f_b2):
    src3 = edge_index[0].astype(jnp.int32).reshape(SC_TILES, STEPS, EB)
    dst3 = edge_index[1].astype(jnp.int32).reshape(SC_TILES, STEPS, EB)
    z2 = (atom_types.astype(jnp.int32) - 1).reshape(N, 1)
    batch2 = batch.astype(jnp.int32).reshape(N, 1)

    emb_pad = jnp.zeros((128, H), jnp.float32).at[:100].set(node_emb)

    ea_lo, ea_hi = _edge_mlp(edge_attr, e_w1, e_b1, e_w2, e_b2)
    x_lo, x_hi = _embed(z2, emb_pad)

    for i in range(NC):
        a_lo, a_hi = _sc_conv(x_lo, x_hi, ea_lo, ea_hi, src3, dst3)
        x_lo, x_hi = _node_mlp(x_lo, x_hi, a_lo, a_hi,
                               c_w1[i], c_b1[i], c_w2[i], c_b2[i],
                               inner_relu=(i < NC - 1))

    return _pool_head(x_lo, x_hi, batch2, f_w1, f_b1, f_w2, f_b2)


# speedup vs baseline: 4.3071x; 1.1046x over previous
import functools

import jax
import jax.numpy as jnp
from jax import lax
from jax.experimental import pallas as pl
from jax.experimental.pallas import tpu as pltpu
from jax.experimental.pallas import tpu_sc as plsc

N = 10000
E = 160000
H = 256
L = 128
ED = 16
NC = 3
G = 64

F = H // 2
SC_TILES = 16
EDGES_PER_TILE = E // SC_TILES
EB = 40
STEPS = EDGES_PER_TILE // EB
ROWS_PER_TILE = 624
ROWS_TAIL = N - SC_TILES * ROWS_PER_TILE


NBUF = 3
MAIN_STEPS = (STEPS // NBUF) * NBUF


def _sc_conv(x_lo, x_hi, ea_lo, ea_hi, ei4):
    mesh = plsc.VectorSubcoreMesh(core_axis_name="c", subcore_axis_name="s",
                                  num_cores=2, num_subcores=16)

    @functools.partial(
        pl.kernel,
        out_type=[jax.ShapeDtypeStruct((N, F), jnp.float32),
                  jax.ShapeDtypeStruct((N, F), jnp.float32)],
        mesh=mesh,
        scratch_types=[
            [pltpu.VMEM((EB,), jnp.int32)] * NBUF,
            [pltpu.VMEM((EB,), jnp.int32)] * NBUF,
            [pltpu.VMEM((EB, F), jnp.float32)] * NBUF,
            [pltpu.VMEM((EB, F), jnp.float32)] * NBUF,
            pltpu.VMEM_SHARED((N, F), jnp.float32),
            [pltpu.SemaphoreType.DMA] * NBUF,
            [pltpu.SemaphoreType.DMA] * NBUF,
            [pltpu.SemaphoreType.DMA] * NBUF,
        ],
    )
    def conv(x_lo_hbm, x_hi_hbm, ea_lo_hbm, ea_hi_hbm, ei_hbm,
             out_lo, out_hi, src_ix, dst_ix, rows, eav, aggr_sh,
             sem_io, sem_sc, sem_six):
        c = lax.axis_index("c")
        s = lax.axis_index("s")

        zero16 = jnp.zeros((16,), jnp.float32)

        @plsc.parallel_loop(0, EB)
        def _(i):
            for j in range(F // 16):
                rows[0][i, pl.ds(j * 16, 16)] = zero16

        row0 = s * ROWS_PER_TILE
        done = 0
        while done < ROWS_PER_TILE:
            cnt = min(EB, ROWS_PER_TILE - done)
            pltpu.sync_copy(rows[0].at[pl.ds(0, cnt)],
                            aggr_sh.at[pl.ds(row0 + done, cnt)])
            done += cnt

        @pl.when(s == SC_TILES - 1)
        def _():
            pltpu.sync_copy(rows[0].at[pl.ds(0, ROWS_TAIL)],
                            aggr_sh.at[pl.ds(SC_TILES * ROWS_PER_TILE,
                                             ROWS_TAIL)])
        plsc.subcore_barrier()

        ebase = s * EDGES_PER_TILE

        def edge_pass(x_hbm, ea_hbm):
            def load_src(b, k):
                pltpu.async_copy(ei_hbm.at[0, s, k], src_ix[b], sem_six[b])

            def wait_six(b):
                pltpu.make_async_copy(ei_hbm.at[0, s, 0], src_ix[b],
                                      sem_six[b]).wait()

            def issue_io(b, k):
                pltpu.async_copy(x_hbm.at[src_ix[b]], rows[b], sem_io[b])
                pltpu.async_copy(ea_hbm.at[pl.ds(ebase + k * EB, EB)],
                                 eav[b], sem_io[b])
                pltpu.async_copy(ei_hbm.at[1, s, k], dst_ix[b], sem_io[b])

            def wait_io(b):
                pltpu.make_async_copy(ea_hbm.at[pl.ds(0, EB)], rows[b],
                                      sem_io[b]).wait()
                pltpu.make_async_copy(ea_hbm.at[pl.ds(0, EB)], eav[b],
                                      sem_io[b]).wait()
                pltpu.make_async_copy(ei_hbm.at[1, s, 0], dst_ix[b],
                                      sem_io[b]).wait()

            def compute(b):
                @plsc.parallel_loop(0, EB, unroll=4)
                def _(i):
                    for j in range(F // 16):
                        sl = pl.ds(j * 16, 16)
                        rows[b][i, sl] = jnp.maximum(
                            rows[b][i, sl] + eav[b][i, sl], 0.0)

            def issue_scatter(b):
                pltpu.async_copy(rows[b], aggr_sh.at[dst_ix[b]],
                                 sem_sc[b], add=True)

            def wait_scatter(b):
                pltpu.make_async_copy(ea_hbm.at[pl.ds(0, EB)], rows[b],
                                      sem_sc[b]).wait()

            def step_fn(k, p, q):
                wait_io(p)

                @pl.when(k + NBUF < STEPS)
                def _():
                    load_src(p, k + NBUF)

                compute(p)
                issue_scatter(p)

                @pl.when((k >= 1) & (k + 2 < STEPS))
                def _():
                    wait_scatter(q)
                    wait_six(q)
                    issue_io(q, k + 2)

            for b in range(NBUF):
                pltpu.sync_copy(ei_hbm.at[0, s, b], src_ix[b])
            for b in range(NBUF):
                issue_io(b, b)

            def tri(i, _):
                k0 = i * NBUF
                for b in range(NBUF):
                    step_fn(k0 + b, b, (b - 1) % NBUF)
                return 0
            lax.fori_loop(0, STEPS // NBUF, tri, 0)

            for k in range(MAIN_STEPS, STEPS):
                p = k % NBUF
                wait_io(p)
                compute(p)
                issue_scatter(p)

            for b in range(NBUF):
                wait_scatter(b)

        @pl.when(c == 0)
        def _():
            edge_pass(x_lo_hbm, ea_lo_hbm)

        @pl.when(c == 1)
        def _():
            edge_pass(x_hi_hbm, ea_hi_hbm)

        plsc.subcore_barrier()

        def copy_out(out_ref):
            pltpu.sync_copy(aggr_sh.at[pl.ds(row0, ROWS_PER_TILE)],
                            out_ref.at[pl.ds(row0, ROWS_PER_TILE)])

            @pl.when(s == SC_TILES - 1)
            def _():
                base = SC_TILES * ROWS_PER_TILE
                pltpu.sync_copy(aggr_sh.at[pl.ds(base, ROWS_TAIL)],
                                out_ref.at[pl.ds(base, ROWS_TAIL)])

        @pl.when(c == 0)
        def _():
            copy_out(out_lo)

        @pl.when(c == 1)
        def _():
            copy_out(out_hi)

    return conv(x_lo, x_hi, ea_lo, ea_hi, ei4)


def _edge_mlp(edge_attr, e_w1, e_b1, e_w2, e_b2):
    BE = 2000

    def body(ea_ref, w1_ref, b1_ref, w2_ref, b2_ref, lo_ref, hi_ref):
        h = jnp.maximum(
            jnp.dot(ea_ref[...].astype(jnp.bfloat16), w1_ref[...],
                    preferred_element_type=jnp.float32)
            + b1_ref[...], 0.0)
        o = jnp.dot(h.astype(jnp.bfloat16), w2_ref[...],
                    preferred_element_type=jnp.float32) + b2_ref[...]
        lo_ref[...] = o[:, :F]
        hi_ref[...] = o[:, F:]

    return pl.pallas_call(
        body,
        grid=(E // BE,),
        in_specs=[
            pl.BlockSpec((BE, ED), lambda i: (i, 0)),
            pl.BlockSpec((ED, H), lambda i: (0, 0)),
            pl.BlockSpec((1, H), lambda i: (0, 0)),
            pl.BlockSpec((H, H), lambda i: (0, 0)),
            pl.BlockSpec((1, H), lambda i: (0, 0)),
        ],
        out_specs=[
            pl.BlockSpec((BE, F), lambda i: (i, 0)),
            pl.BlockSpec((BE, F), lambda i: (i, 0)),
        ],
        out_shape=[jax.ShapeDtypeStruct((E, F), jnp.float32),
                   jax.ShapeDtypeStruct((E, F), jnp.float32)],
    )(edge_attr, e_w1.astype(jnp.bfloat16), e_b1.reshape(1, H),
      e_w2.astype(jnp.bfloat16), e_b2.reshape(1, H))


def _embed(z2, emb_pad):
    BN = 2000
    K = emb_pad.shape[0]

    def body(z_ref, emb_ref, lo_ref, hi_ref):
        z = z_ref[...]
        ids = lax.broadcasted_iota(jnp.int32, (BN, K), 1)
        onehot = jnp.where(ids == z, 1.0, 0.0).astype(jnp.float32)
        x = jnp.dot(onehot, emb_ref[...], preferred_element_type=jnp.float32)
        lo_ref[...] = x[:, :F]
        hi_ref[...] = x[:, F:]

    return pl.pallas_call(
        body,
        grid=(N // BN,),
        in_specs=[
            pl.BlockSpec((BN, 1), lambda i: (i, 0)),
            pl.BlockSpec((K, H), lambda i: (0, 0)),
        ],
        out_specs=[
            pl.BlockSpec((BN, F), lambda i: (i, 0)),
            pl.BlockSpec((BN, F), lambda i: (i, 0)),
        ],
        out_shape=[jax.ShapeDtypeStruct((N, F), jnp.float32),
                   jax.ShapeDtypeStruct((N, F), jnp.float32)],
    )(z2, emb_pad)


def _node_mlp(x_lo, x_hi, a_lo, a_hi, w1, b1, w2, b2, inner_relu):
    BN = 2000

    def body(xl_ref, xh_ref, al_ref, ah_ref, w1_ref, b1_ref, w2_ref, b2_ref,
             ol_ref, oh_ref):
        x = jnp.concatenate([xl_ref[...], xh_ref[...]], axis=1)
        a = jnp.concatenate([al_ref[...], ah_ref[...]], axis=1)
        h = x + a
        h = jnp.maximum(
            jnp.dot(h, w1_ref[...], preferred_element_type=jnp.float32)
            + b1_ref[...], 0.0)
        h = jnp.dot(h, w2_ref[...], preferred_element_type=jnp.float32) + b2_ref[...]
        if inner_relu:
            h = jnp.maximum(h, 0.0)
        h = h + x
        ol_ref[...] = h[:, :F]
        oh_ref[...] = h[:, F:]

    return pl.pallas_call(
        body,
        grid=(N // BN,),
        in_specs=[
            pl.BlockSpec((BN, F), lambda i: (i, 0)),
            pl.BlockSpec((BN, F), lambda i: (i, 0)),
            pl.BlockSpec((BN, F), lambda i: (i, 0)),
            pl.BlockSpec((BN, F), lambda i: (i, 0)),
            pl.BlockSpec((H, H), lambda i: (0, 0)),
            pl.BlockSpec((1, H), lambda i: (0, 0)),
            pl.BlockSpec((H, H), lambda i: (0, 0)),
            pl.BlockSpec((1, H), lambda i: (0, 0)),
        ],
        out_specs=[
            pl.BlockSpec((BN, F), lambda i: (i, 0)),
            pl.BlockSpec((BN, F), lambda i: (i, 0)),
        ],
        out_shape=[jax.ShapeDtypeStruct((N, F), jnp.float32),
                   jax.ShapeDtypeStruct((N, F), jnp.float32)],
    )(x_lo, x_hi, a_lo, a_hi, w1, b1.reshape(1, H), w2, b2.reshape(1, H))


def _pool_head(x_lo, x_hi, batch2, f_w1, f_b1, f_w2, f_b2):
    BN = 2000
    NBLK = N // BN

    def body(xl_ref, xh_ref, b_ref, w1_ref, b1_ref, w2_ref, b2_ref, out_ref,
             pooled_ref):
        blk = pl.program_id(0)

        @pl.when(blk == 0)
        def _():
            pooled_ref[...] = jnp.full((G, H), -jnp.inf, jnp.float32)

        x = jnp.concatenate([xl_ref[...], xh_ref[...]], axis=1)
        b = b_ref[...]

        def grp(g, _):
            m = jnp.where(b == g, x, -jnp.inf)
            mx = jnp.max(m, axis=0).reshape(1, H)
            cur = pooled_ref[pl.ds(g, 1), :]
            pooled_ref[pl.ds(g, 1), :] = jnp.maximum(cur, mx)
            return 0
        lax.fori_loop(b_ref[0, 0], b_ref[BN - 1, 0] + 1, grp, 0)

        @pl.when(blk == NBLK - 1)
        def _():
            p = pooled_ref[...]
            h = jnp.maximum(
                jnp.dot(p, w1_ref[...], preferred_element_type=jnp.float32)
                + b1_ref[...], 0.0)
            out_ref[...] = (
                jnp.dot(h, w2_ref[...], preferred_element_type=jnp.float32)
                + b2_ref[...])

    return pl.pallas_call(
        body,
        grid=(NBLK,),
        in_specs=[
            pl.BlockSpec((BN, F), lambda i: (i, 0)),
            pl.BlockSpec((BN, F), lambda i: (i, 0)),
            pl.BlockSpec((BN, 1), lambda i: (i, 0)),
            pl.BlockSpec((H, H), lambda i: (0, 0)),
            pl.BlockSpec((1, H), lambda i: (0, 0)),
            pl.BlockSpec((H, L), lambda i: (0, 0)),
            pl.BlockSpec((1, L), lambda i: (0, 0)),
        ],
        out_specs=pl.BlockSpec((G, L), lambda i: (0, 0)),
        out_shape=jax.ShapeDtypeStruct((G, L), jnp.float32),
        scratch_shapes=[pltpu.VMEM((G, H), jnp.float32)],
    )(x_lo, x_hi, batch2, f_w1, f_b1.reshape(1, H), f_w2, f_b2.reshape(1, L))


def kernel(atom_types, edge_index, edge_attr, batch, node_emb,
           e_w1, e_b1, e_w2, e_b2,
           c_w1, c_b1, c_w2, c_b2,
           f_w1, f_b1, f_w2, f_b2):
    ei4 = edge_index.astype(jnp.int32).reshape(2, SC_TILES, STEPS, EB)
    z2 = (atom_types.astype(jnp.int32) - 1).reshape(N, 1)
    batch2 = batch.astype(jnp.int32).reshape(N, 1)

    emb_pad = jnp.zeros((128, H), jnp.float32).at[:100].set(node_emb)

    ea_lo, ea_hi = _edge_mlp(edge_attr, e_w1, e_b1, e_w2, e_b2)
    x_lo, x_hi = _embed(z2, emb_pad)

    for i in range(NC):
        a_lo, a_hi = _sc_conv(x_lo, x_hi, ea_lo, ea_hi, ei4)
        x_lo, x_hi = _node_mlp(x_lo, x_hi, a_lo, a_hi,
                               c_w1[i], c_b1[i], c_w2[i], c_b2[i],
                               inner_relu=(i < NC - 1))

    return _pool_head(x_lo, x_hi, batch2, f_w1, f_b1, f_w2, f_b2)
